# baseline (speedup 1.0000x reference)
import functools

import jax
import jax.numpy as jnp
from jax import lax
from jax.experimental import pallas as pl
from jax.experimental.pallas import tpu as pltpu
from jax.experimental.pallas import tpu_sc as plsc

_N = 10000
_E = 320000
_D = 128
_C = 47

_NC = 2
_NS = 16
_NW = _NC * _NS
_CH = 128
_ITERS = 80
_EPAD = _NW * _ITERS * _CH
_NP = 10112
_RPT = _NP // _NS
_DEGW = 16

_mesh = plsc.VectorSubcoreMesh(
    core_axis_name="c", subcore_axis_name="s", num_cores=_NC, num_subcores=_NS
)


def _wid(cid, sid):
    return sid * _NC + cid


@functools.partial(
    pl.kernel,
    out_type=jax.ShapeDtypeStruct((_NC, _NP, _D), jnp.float32),
    mesh=_mesh,
    scratch_types=[
        pltpu.VMEM((_ITERS, _CH), jnp.int32),
        pltpu.VMEM((_CH,), jnp.int32),
        pltpu.VMEM((_CH,), jnp.int32),
        pltpu.VMEM((_CH,), jnp.int32),
        pltpu.VMEM((_CH,), jnp.int32),
        pltpu.VMEM((_CH, _D), jnp.float32),
        pltpu.VMEM((_CH, _D), jnp.float32),
        pltpu.VMEM_SHARED((_NP, _D), jnp.float32),
        pltpu.SemaphoreType.DMA,
        pltpu.SemaphoreType.DMA,
        pltpu.SemaphoreType.DMA,
        pltpu.SemaphoreType.DMA,
        pltpu.SemaphoreType.DMA,
        pltpu.SemaphoreType.DMA,
        pltpu.SemaphoreType.DMA,
        pltpu.SemaphoreType.DMA,
    ],
)
def _sc_agg(h_hbm, src2_hbm, dst2_hbm, zeros_hbm, out_hbm,
            dst_v, sb0, sb1, sb2, sb3, rows_a, rows_b, acc_sh,
            si0, si1, si2, si3, semg_a, semg_b, sems_a, sems_b):
    cid = lax.axis_index("c")
    sid = lax.axis_index("s")
    r0 = sid * _RPT
    pltpu.sync_copy(zeros_hbm.at[pl.ds(r0, _RPT)], acc_sh.at[pl.ds(r0, _RPT)])
    wid = _wid(cid, sid)
    pltpu.sync_copy(dst2_hbm.at[wid], dst_v)
    plsc.subcore_barrier()

    sbs = (sb0, sb1, sb2, sb3)
    sis = (si0, si1, si2, si3)
    rows = (rows_a, rows_b)
    semg = (semg_a, semg_b)
    sems = (sems_a, sems_b)

    def il(j, t):
        pltpu.async_copy(src2_hbm.at[wid, j], sbs[t], sis[t])

    def wait_il(t):
        pltpu.make_async_copy(src2_hbm.at[0, 0], sbs[t], sis[t]).wait()

    def sg(t):
        pltpu.async_copy(h_hbm.at[sbs[t]], rows[t % 2], semg[t % 2])

    def wait_g(t):
        pltpu.make_async_copy(
            h_hbm.at[pl.ds(0, _CH)], rows[t % 2], semg[t % 2]
        ).wait()

    def scat(j, t):
        pltpu.async_copy(
            rows[t % 2], acc_sh.at[dst_v.at[j]], sems[t % 2], add=True
        )

    def wait_sc(t):
        pltpu.make_async_copy(
            rows[t % 2], acc_sh.at[pl.ds(0, _CH)], sems[t % 2]
        ).wait()

    def chunk(j, t, first, prefetch):
        if not first:
            wait_sc(t)
        wait_il(t)
        sg(t)
        wait_g(t)
        scat(j, t)
        if prefetch:
            il(j + 4, t)

    il(0, 0)
    il(1, 1)
    il(2, 2)
    il(3, 3)

    for t in range(4):
        chunk(t, t, first=t < 2, prefetch=True)

    def grp(g, carry):
        j0 = 4 * g
        for t in range(4):
            chunk(j0 + t, t, first=False, prefetch=True)
        return carry

    lax.fori_loop(1, _ITERS // 4 - 1, grp, 0)

    j0 = _ITERS - 4
    for t in range(4):
        chunk(j0 + t, t, first=False, prefetch=False)
    wait_sc(0)
    wait_sc(1)

    plsc.subcore_barrier()
    pltpu.sync_copy(acc_sh.at[pl.ds(r0, _RPT)], out_hbm.at[cid, pl.ds(r0, _RPT)])


_BN = 1000


def _tc_body(relu, h_b, a0_b, a1_b, d0_b, d1_b, ws_b, wn_b, b_b, o_b):
    deg = jnp.maximum(d0_b[:, 0:1] + d1_b[:, 0:1], 1.0)
    hn = (a0_b[...] + a1_b[...]) / deg
    o = (
        jnp.dot(h_b[...], ws_b[...], preferred_element_type=jnp.float32)
        + jnp.dot(hn, wn_b[...], preferred_element_type=jnp.float32)
        + b_b[...]
    )
    if relu:
        o = jnp.maximum(o, 0.0)
    o_b[...] = o


def _tc_layer(h, a0, a1, d0, d1, ws, wn, b, relu):
    body = functools.partial(_tc_body, relu)
    return pl.pallas_call(
        body,
        grid=(_N // _BN,),
        in_specs=[
            pl.BlockSpec((_BN, _D), lambda i: (i, 0)),
            pl.BlockSpec((_BN, _D), lambda i: (i, 0)),
            pl.BlockSpec((_BN, _D), lambda i: (i, 0)),
            pl.BlockSpec((_BN, _DEGW), lambda i: (i, 0)),
            pl.BlockSpec((_BN, _DEGW), lambda i: (i, 0)),
            pl.BlockSpec((_D, _D), lambda i: (0, 0)),
            pl.BlockSpec((_D, _D), lambda i: (0, 0)),
            pl.BlockSpec((1, _D), lambda i: (0, 0)),
        ],
        out_specs=pl.BlockSpec((_BN, _D), lambda i: (i, 0)),
        out_shape=jax.ShapeDtypeStruct((_N, _D), jnp.float32),
    )(h, a0, a1, d0, d1, ws, wn, b.reshape(1, _D))


def _pad_cols(w):
    return jnp.pad(w, ((0, 0), (0, _D - w.shape[1])))


def kernel(x, edge_index, W_self0, W_neigh0, b0, W_self1, W_neigh1, b1,
           W_self2, W_neigh2, b2):
    npad = _EPAD - _E
    pad_ids = jnp.arange(npad, dtype=jnp.int32)
    src = jnp.concatenate(
        [edge_index[0], pad_ids % _N]
    ).reshape(_NW, _ITERS, _CH)
    dst = jnp.concatenate(
        [edge_index[1], _N + pad_ids % (_NP - _N)]
    ).reshape(_NW, _ITERS, _CH)
    z_nd = jnp.zeros((_NP, _D), jnp.float32)

    src_z = jnp.zeros_like(src)
    degp = _sc_agg(jnp.ones((_N, _D), jnp.float32), src_z, dst, z_nd)
    d0, d1 = degp[0, :_N, :_DEGW], degp[1, :_N, :_DEGW]
    z_nd, _ = lax.optimization_barrier((z_nd, degp))

    a = _sc_agg(x, src, dst, z_nd)
    h1 = _tc_layer(x, a[0, :_N], a[1, :_N], d0, d1, W_self0, W_neigh0, b0, relu=True)

    a = _sc_agg(h1, src, dst, z_nd)
    h2 = _tc_layer(h1, a[0, :_N], a[1, :_N], d0, d1, W_self1, W_neigh1, b1, relu=True)

    a = _sc_agg(h2, src, dst, z_nd)
    out = _tc_layer(
        h2, a[0, :_N], a[1, :_N], d0, d1,
        _pad_cols(W_self2), _pad_cols(W_neigh2),
        jnp.pad(b2, (0, _D - _C)), relu=False,
    )
    return out[:, :_C]


# speedup vs baseline: 20.4275x; 20.4275x over previous
import functools

import jax
import jax.numpy as jnp
from jax import lax
from jax.experimental import pallas as pl
from jax.experimental.pallas import tpu as pltpu
from jax.experimental.pallas import tpu_sc as plsc

_N = 10000
_E = 320000
_D = 128
_C = 47

_NC = 2
_NS = 16
_NW = _NC * _NS
_CH = 128
_ITERS = 80
_EPAD = _NW * _ITERS * _CH
_NP = 10112
_RPT = _NP // _NS
_DEGW = 16

_mesh = plsc.VectorSubcoreMesh(
    core_axis_name="c", subcore_axis_name="s", num_cores=_NC, num_subcores=_NS
)


def _wid(cid, sid):
    return sid * _NC + cid


@functools.partial(
    pl.kernel,
    out_type=jax.ShapeDtypeStruct((_NC, _NP, _D), jnp.float32),
    mesh=_mesh,
    scratch_types=[
        pltpu.VMEM((_ITERS, _CH), jnp.int32),
        pltpu.VMEM((_CH,), jnp.int32),
        pltpu.VMEM((_CH,), jnp.int32),
        pltpu.VMEM((_CH,), jnp.int32),
        pltpu.VMEM((_CH,), jnp.int32),
        pltpu.VMEM((_CH, _D), jnp.float32),
        pltpu.VMEM((_CH, _D), jnp.float32),
        pltpu.VMEM_SHARED((_NP, _D), jnp.float32),
        pltpu.SemaphoreType.DMA,
        pltpu.SemaphoreType.DMA,
        pltpu.SemaphoreType.DMA,
        pltpu.SemaphoreType.DMA,
        pltpu.SemaphoreType.DMA,
        pltpu.SemaphoreType.DMA,
        pltpu.SemaphoreType.DMA,
        pltpu.SemaphoreType.DMA,
    ],
)
def _sc_agg(h_hbm, src2_hbm, dst2_hbm, zeros_hbm, out_hbm,
            dst_v, sb0, sb1, sb2, sb3, rows_a, rows_b, acc_sh,
            si0, si1, si2, si3, semg_a, semg_b, sems_a, sems_b):
    cid = lax.axis_index("c")
    sid = lax.axis_index("s")
    r0 = sid * _RPT
    pltpu.sync_copy(zeros_hbm.at[pl.ds(r0, _RPT)], acc_sh.at[pl.ds(r0, _RPT)])
    wid = _wid(cid, sid)
    pltpu.sync_copy(dst2_hbm.at[wid], dst_v)
    plsc.subcore_barrier()

    sbs = (sb0, sb1, sb2, sb3)
    sis = (si0, si1, si2, si3)
    rows = (rows_a, rows_b)
    semg = (semg_a, semg_b)
    sems = (sems_a, sems_b)

    def il(j, t):
        pltpu.async_copy(src2_hbm.at[wid, j], sbs[t], sis[t])

    def wait_il(t):
        pltpu.make_async_copy(src2_hbm.at[0, 0], sbs[t], sis[t]).wait()

    def sg(t):
        pltpu.async_copy(h_hbm.at[sbs[t]], rows[t % 2], semg[t % 2])

    def wait_g(t):
        pltpu.make_async_copy(
            h_hbm.at[pl.ds(0, _CH)], rows[t % 2], semg[t % 2]
        ).wait()

    def scat(j, t):
        pltpu.async_copy(
            rows[t % 2], acc_sh.at[dst_v.at[j]], sems[t % 2], add=True
        )

    def wait_sc(t):
        pltpu.make_async_copy(
            rows[t % 2], acc_sh.at[pl.ds(0, _CH)], sems[t % 2]
        ).wait()

    def chunk(j, t, first, prefetch):
        if not first:
            wait_sc(t)
        wait_il(t)
        sg(t)
        wait_g(t)
        scat(j, t)
        if prefetch:
            il(j + 4, t)

    il(0, 0)
    il(1, 1)
    il(2, 2)
    il(3, 3)

    for t in range(4):
        chunk(t, t, first=t < 2, prefetch=True)

    def grp(g, carry):
        j0 = 4 * g
        for t in range(4):
            chunk(j0 + t, t, first=False, prefetch=True)
        return carry

    lax.fori_loop(1, _ITERS // 4 - 1, grp, 0)

    j0 = _ITERS - 4
    for t in range(4):
        chunk(j0 + t, t, first=False, prefetch=False)
    wait_sc(0)
    wait_sc(1)

    plsc.subcore_barrier()
    pltpu.sync_copy(acc_sh.at[pl.ds(r0, _RPT)], out_hbm.at[cid, pl.ds(r0, _RPT)])


_BN = 1000


def _tc_body(relu, h_b, a0_b, a1_b, d0_b, d1_b, ws_b, wn_b, b_b, o_b):
    deg = jnp.maximum(d0_b[:, 0:1] + d1_b[:, 0:1], 1.0)
    hn = (a0_b[...] + a1_b[...]) / deg
    o = (
        jnp.dot(h_b[...], ws_b[...], preferred_element_type=jnp.float32)
        + jnp.dot(hn, wn_b[...], preferred_element_type=jnp.float32)
        + b_b[...]
    )
    if relu:
        o = jnp.maximum(o, 0.0)
    o_b[...] = o


def _tc_layer(h, a0, a1, d0, d1, ws, wn, b, relu):
    body = functools.partial(_tc_body, relu)
    return pl.pallas_call(
        body,
        grid=(_N // _BN,),
        in_specs=[
            pl.BlockSpec((_BN, _D), lambda i: (i, 0)),
            pl.BlockSpec((_BN, _D), lambda i: (i, 0)),
            pl.BlockSpec((_BN, _D), lambda i: (i, 0)),
            pl.BlockSpec((_BN, _DEGW), lambda i: (i, 0)),
            pl.BlockSpec((_BN, _DEGW), lambda i: (i, 0)),
            pl.BlockSpec((_D, _D), lambda i: (0, 0)),
            pl.BlockSpec((_D, _D), lambda i: (0, 0)),
            pl.BlockSpec((1, _D), lambda i: (0, 0)),
        ],
        out_specs=pl.BlockSpec((_BN, _D), lambda i: (i, 0)),
        out_shape=jax.ShapeDtypeStruct((_N, _D), jnp.float32),
    )(h, a0, a1, d0, d1, ws, wn, b.reshape(1, _D))


def _pad_cols(w):
    return jnp.pad(w, ((0, 0), (0, _D - w.shape[1])))


def kernel(x, edge_index, W_self0, W_neigh0, b0, W_self1, W_neigh1, b1,
           W_self2, W_neigh2, b2):
    npad = _EPAD - _E
    pad_ids = jnp.arange(npad, dtype=jnp.int32)
    src = jnp.concatenate(
        [edge_index[0], pad_ids % _N]
    ).reshape(_NW, _ITERS, _CH)
    dst = jnp.concatenate(
        [edge_index[1], _N + pad_ids % (_NP - _N)]
    ).reshape(_NW, _ITERS, _CH)
    z_nd = jnp.zeros((_NP, _D), jnp.float32)

    degp = _sc_agg(jnp.ones((_N, _D), jnp.float32), src, dst, z_nd)
    d0, d1 = degp[0, :_N, :_DEGW], degp[1, :_N, :_DEGW]
    z_nd, _ = lax.optimization_barrier((z_nd, degp))

    a = _sc_agg(x, src, dst, z_nd)
    h1 = _tc_layer(x, a[0, :_N], a[1, :_N], d0, d1, W_self0, W_neigh0, b0, relu=True)

    a = _sc_agg(h1, src, dst, z_nd)
    h2 = _tc_layer(h1, a[0, :_N], a[1, :_N], d0, d1, W_self1, W_neigh1, b1, relu=True)

    a = _sc_agg(h2, src, dst, z_nd)
    out = _tc_layer(
        h2, a[0, :_N], a[1, :_N], d0, d1,
        _pad_cols(W_self2), _pad_cols(W_neigh2),
        jnp.pad(b2, (0, _D - _C)), relu=False,
    )
    return out[:, :_C]


# speedup vs baseline: 23.8248x; 1.1663x over previous
import functools

import jax
import jax.numpy as jnp
from jax import lax
from jax.experimental import pallas as pl
from jax.experimental.pallas import tpu as pltpu
from jax.experimental.pallas import tpu_sc as plsc

_N = 10000
_E = 320000
_D = 128
_C = 47

_NC = 2
_NS = 16
_NW = _NC * _NS
_CH = 128
_ITERS = 80
_EPAD = _NW * _ITERS * _CH
_NP = 10112
_RPT = _NP // _NS
_DEGW = 16

_mesh = plsc.VectorSubcoreMesh(
    core_axis_name="c", subcore_axis_name="s", num_cores=_NC, num_subcores=_NS
)


def _wid(cid, sid):
    return sid * _NC + cid


@functools.partial(
    pl.kernel,
    out_type=jax.ShapeDtypeStruct((_NC, _NP, _D), jnp.float32),
    mesh=_mesh,
    scratch_types=[
        pltpu.VMEM((_ITERS, _CH), jnp.int32),
        pltpu.VMEM((_CH,), jnp.int32),
        pltpu.VMEM((_CH,), jnp.int32),
        pltpu.VMEM((_CH,), jnp.int32),
        pltpu.VMEM((_CH,), jnp.int32),
        pltpu.VMEM((_CH, _D), jnp.float32),
        pltpu.VMEM((_CH, _D), jnp.float32),
        pltpu.VMEM_SHARED((_NP, _D), jnp.float32),
        pltpu.SemaphoreType.DMA,
        pltpu.SemaphoreType.DMA,
        pltpu.SemaphoreType.DMA,
        pltpu.SemaphoreType.DMA,
        pltpu.SemaphoreType.DMA,
        pltpu.SemaphoreType.DMA,
        pltpu.SemaphoreType.DMA,
        pltpu.SemaphoreType.DMA,
    ],
)
def _sc_agg(h_hbm, src2_hbm, dst2_hbm, zeros_hbm, out_hbm,
            dst_v, sb0, sb1, sb2, sb3, rows_a, rows_b, acc_sh,
            si0, si1, si2, si3, semg_a, semg_b, sems_a, sems_b):
    cid = lax.axis_index("c")
    sid = lax.axis_index("s")
    r0 = sid * _RPT
    pltpu.sync_copy(zeros_hbm.at[pl.ds(r0, _RPT)], acc_sh.at[pl.ds(r0, _RPT)])
    wid = _wid(cid, sid)
    pltpu.sync_copy(dst2_hbm.at[wid], dst_v)
    plsc.subcore_barrier()

    sbs = (sb0, sb1, sb2, sb3)
    sis = (si0, si1, si2, si3)
    rows = (rows_a, rows_b)
    semg = (semg_a, semg_b)
    sems = (sems_a, sems_b)

    def il(j, t):
        pltpu.async_copy(src2_hbm.at[wid, j], sbs[t], sis[t])

    def wait_il(t):
        pltpu.make_async_copy(src2_hbm.at[0, 0], sbs[t], sis[t]).wait()

    def sg(t):
        pltpu.async_copy(h_hbm.at[sbs[t]], rows[t % 2], semg[t % 2])

    def wait_g(t):
        pltpu.make_async_copy(
            h_hbm.at[pl.ds(0, _CH)], rows[t % 2], semg[t % 2]
        ).wait()

    def scat(j, t):
        pltpu.async_copy(
            rows[t % 2], acc_sh.at[dst_v.at[j]], sems[t % 2], add=True
        )

    def wait_sc(t):
        pltpu.make_async_copy(
            rows[t % 2], acc_sh.at[pl.ds(0, _CH)], sems[t % 2]
        ).wait()

    def start(j, t, first=False):
        if not first:
            wait_sc(t)
        wait_il(t)
        sg(t)

    def fin(j, t, prefetch=True):
        wait_g(t)
        scat(j, t)
        if prefetch:
            il(j + 4, t)

    il(0, 0)
    il(1, 1)
    il(2, 2)
    il(3, 3)

    start(0, 0, first=True)
    start(1, 1, first=True)
    fin(0, 0)
    start(2, 2)
    fin(1, 1)
    start(3, 3)

    def grp(g, carry):
        j0 = 4 * g
        for t in range(4):
            j = j0 + t
            fin(j - 2, (t - 2) % 4)
            start(j, t)
        return carry

    lax.fori_loop(1, _ITERS // 4 - 1, grp, 0)

    j0 = _ITERS - 4
    fin(j0 - 2, 2)
    start(j0, 0)
    fin(j0 - 1, 3)
    start(j0 + 1, 1)
    fin(j0, 0, prefetch=False)
    start(j0 + 2, 2)
    fin(j0 + 1, 1, prefetch=False)
    start(j0 + 3, 3)
    fin(j0 + 2, 2, prefetch=False)
    fin(j0 + 3, 3, prefetch=False)
    wait_sc(0)
    wait_sc(1)

    plsc.subcore_barrier()
    pltpu.sync_copy(acc_sh.at[pl.ds(r0, _RPT)], out_hbm.at[cid, pl.ds(r0, _RPT)])


_BN = 1000


def _tc_body(relu, h_b, a0_b, a1_b, d0_b, d1_b, ws_b, wn_b, b_b, o_b):
    deg = jnp.maximum(d0_b[:, 0:1] + d1_b[:, 0:1], 1.0)
    hn = (a0_b[...] + a1_b[...]) / deg
    o = (
        jnp.dot(h_b[...], ws_b[...], preferred_element_type=jnp.float32)
        + jnp.dot(hn, wn_b[...], preferred_element_type=jnp.float32)
        + b_b[...]
    )
    if relu:
        o = jnp.maximum(o, 0.0)
    o_b[...] = o


def _tc_layer(h, a0, a1, d0, d1, ws, wn, b, relu):
    body = functools.partial(_tc_body, relu)
    return pl.pallas_call(
        body,
        grid=(_N // _BN,),
        in_specs=[
            pl.BlockSpec((_BN, _D), lambda i: (i, 0)),
            pl.BlockSpec((_BN, _D), lambda i: (i, 0)),
            pl.BlockSpec((_BN, _D), lambda i: (i, 0)),
            pl.BlockSpec((_BN, _DEGW), lambda i: (i, 0)),
            pl.BlockSpec((_BN, _DEGW), lambda i: (i, 0)),
            pl.BlockSpec((_D, _D), lambda i: (0, 0)),
            pl.BlockSpec((_D, _D), lambda i: (0, 0)),
            pl.BlockSpec((1, _D), lambda i: (0, 0)),
        ],
        out_specs=pl.BlockSpec((_BN, _D), lambda i: (i, 0)),
        out_shape=jax.ShapeDtypeStruct((_N, _D), jnp.float32),
    )(h, a0, a1, d0, d1, ws, wn, b.reshape(1, _D))


def _pad_cols(w):
    return jnp.pad(w, ((0, 0), (0, _D - w.shape[1])))


def kernel(x, edge_index, W_self0, W_neigh0, b0, W_self1, W_neigh1, b1,
           W_self2, W_neigh2, b2):
    npad = _EPAD - _E
    pad_ids = jnp.arange(npad, dtype=jnp.int32)
    src = jnp.concatenate(
        [edge_index[0], pad_ids % _N]
    ).reshape(_NW, _ITERS, _CH)
    dst = jnp.concatenate(
        [edge_index[1], _N + pad_ids % (_NP - _N)]
    ).reshape(_NW, _ITERS, _CH)
    z_nd = jnp.zeros((_NP, _D), jnp.float32)

    degp = _sc_agg(jnp.ones((_N, _D), jnp.float32), src, dst, z_nd)
    d0, d1 = degp[0, :_N, :_DEGW], degp[1, :_N, :_DEGW]
    z_nd, _ = lax.optimization_barrier((z_nd, degp))

    a = _sc_agg(x, src, dst, z_nd)
    h1 = _tc_layer(x, a[0, :_N], a[1, :_N], d0, d1, W_self0, W_neigh0, b0, relu=True)

    a = _sc_agg(h1, src, dst, z_nd)
    h2 = _tc_layer(h1, a[0, :_N], a[1, :_N], d0, d1, W_self1, W_neigh1, b1, relu=True)

    a = _sc_agg(h2, src, dst, z_nd)
    out = _tc_layer(
        h2, a[0, :_N], a[1, :_N], d0, d1,
        _pad_cols(W_self2), _pad_cols(W_neigh2),
        jnp.pad(b2, (0, _D - _C)), relu=False,
    )
    return out[:, :_C]
